# baseline (speedup 1.0000x reference)
import functools

import jax
import jax.numpy as jnp
from jax import lax
from jax.experimental import pallas as pl
from jax.experimental.pallas import tpu as pltpu
from jax.experimental.pallas import tpu_sc as plsc

_DELTA_VAR = 0.5
_DELTA_DST = 1.5
_A = 1.0
_B = 1.0
_R = 0.001
_NL = 16
_E = 16
_EH = 8
_LANES = 16
_NW = 32
_RB = 8
_W = 512


def _nrsqrt(x):
    i = lax.bitcast_convert_type(x, jnp.int32)
    i = jnp.int32(0x5F3759DF) - lax.shift_right_arithmetic(i, 1)
    y = lax.bitcast_convert_type(i, jnp.float32)
    for _ in range(3):
        y = y * (1.5 - 0.5 * x * y * y)
    return y


@functools.lru_cache(maxsize=None)
def _make_pass1(Bsz, H, unroll):
    RPT = H // _NW
    nrb = RPT // _RB
    G = _RB * _W // _LANES
    nflat = Bsz * nrb * 2
    ACC = Bsz * _E * _NL * _LANES
    CNT = Bsz * _NL * _LANES
    mesh = plsc.VectorSubcoreMesh(core_axis_name="c", subcore_axis_name="s")
    out_types = (
        jax.ShapeDtypeStruct((_NW, ACC), jnp.float32),
        jax.ShapeDtypeStruct((_NW, CNT), jnp.float32),
    )

    @functools.partial(
        pl.kernel,
        out_type=out_types,
        mesh=mesh,
        compiler_params=pltpu.CompilerParams(needs_layout_passes=False),
        scratch_types=[
            pltpu.VMEM((2, _EH, _RB, _W), jnp.float32),
            pltpu.VMEM((2, _RB, _W), jnp.int32),
            pltpu.VMEM((ACC,), jnp.float32),
            pltpu.VMEM((CNT,), jnp.float32),
            pltpu.SemaphoreType.DMA,
            pltpu.SemaphoreType.DMA,
            pltpu.SemaphoreType.DMA,
            pltpu.SemaphoreType.DMA,
        ],
    )
    def pass1(emb_hbm, lab_hbm, zeros_hbm, sums_hbm, cnt_hbm,
              embv, labv, acc, cnt, sem0, sem1, lsem0, lsem1):
        wid = lax.axis_index("s") * 2 + lax.axis_index("c")
        iota = lax.iota(jnp.int32, _LANES)
        ones = jnp.ones((_LANES,), jnp.float32)
        row0 = wid * RPT
        esems = (sem0, sem1)
        lsems = (lsem0, lsem1)

        def start_emb(idx):
            rbg, esub = divmod(idx, 2)
            b, rb = divmod(rbg, nrb)
            slot = idx % 2
            r = row0 + rb * _RB
            return pltpu.async_copy(
                emb_hbm.at[b, pl.ds(esub * _EH, _EH), pl.ds(r, _RB), :],
                embv.at[slot], esems[slot])

        def start_lab(rbg):
            b, rb = divmod(rbg, nrb)
            slot = rbg % 2
            r = row0 + rb * _RB
            return pltpu.async_copy(
                lab_hbm.at[b, pl.ds(r, _RB), :], labv.at[slot], lsems[slot])

        pltpu.sync_copy(zeros_hbm.at[pl.ds(0, ACC)], acc)
        pltpu.sync_copy(zeros_hbm.at[pl.ds(0, CNT)], cnt)

        epend = {0: start_emb(0)}
        lpend = {0: start_lab(0)}
        for idx in range(nflat):
            rbg, esub = divmod(idx, 2)
            b, _rb = divmod(rbg, nrb)
            slot = idx % 2
            lslot = rbg % 2
            epend.pop(idx).wait()
            if esub == 0:
                lpend.pop(rbg).wait()
            if idx + 1 < nflat:
                epend[idx + 1] = start_emb(idx + 1)
                if (idx + 1) % 2 == 0 and rbg + 1 < Bsz * nrb:
                    lpend[rbg + 1] = start_lab(rbg + 1)

            @plsc.parallel_loop(0, G, unroll=unroll)
            def body(g):
                r = lax.shift_right_logical(g, 5)
                col = lax.shift_left(lax.bitwise_and(g, 31), 4)
                lab = labv[lslot, r, pl.ds(col, _LANES)]
                lane_base = lab * _LANES + iota + b * (_E * _NL * _LANES)
                for e in range(_EH):
                    x = embv[slot, e, r, pl.ds(col, _LANES)]
                    plsc.addupdate_scatter(
                        acc,
                        [lane_base + (esub * _EH + e) * (_NL * _LANES)], x)
                if esub == 0:
                    plsc.addupdate_scatter(
                        cnt, [lab * _LANES + iota + b * (_NL * _LANES)], ones)

        pltpu.sync_copy(acc, sums_hbm.at[wid])
        pltpu.sync_copy(cnt, cnt_hbm.at[wid])

    return pass1


@functools.lru_cache(maxsize=None)
def _make_pass2(Bsz, H, unroll):
    RPT = H // _NW
    nrb = RPT // _RB
    G = _RB * _W // _LANES
    CPX = _RB * _W
    nflat = Bsz * nrb * 2
    HACC = Bsz * _NL * _LANES
    mesh = plsc.VectorSubcoreMesh(core_axis_name="c", subcore_axis_name="s")
    out_types = (
        jax.ShapeDtypeStruct((_NW, HACC), jnp.float32),
        jax.ShapeDtypeStruct((Bsz, 3, _LANES), jnp.float32),
    )

    @functools.partial(
        pl.kernel,
        out_type=out_types,
        mesh=mesh,
        compiler_params=pltpu.CompilerParams(needs_layout_passes=False),
        scratch_types=[
            pltpu.VMEM((2, _EH, _RB, _W), jnp.float32),
            pltpu.VMEM((2, _RB, _W), jnp.int32),
            pltpu.VMEM((Bsz * _NL * _E,), jnp.float32),
            pltpu.VMEM((Bsz, _E, _NL), jnp.float32),
            pltpu.VMEM((Bsz * _NL,), jnp.float32),
            pltpu.VMEM((Bsz, _NL), jnp.float32),
            pltpu.VMEM((HACC,), jnp.float32),
            pltpu.VMEM((CPX,), jnp.float32),
            pltpu.VMEM((CPX,), jnp.float32),
            pltpu.VMEM((3, _LANES), jnp.float32),
            pltpu.SemaphoreType.DMA,
            pltpu.SemaphoreType.DMA,
            pltpu.SemaphoreType.DMA,
            pltpu.SemaphoreType.DMA,
        ],
    )
    def pass2(emb_hbm, lab_hbm, zeros_hbm, ctr_hbm, ctrT_hbm, csq_hbm,
              prs_hbm, hout_hbm, pout_hbm,
              embv, labv, ctrv, ctrTv, csqv, prsv, hacc, cxv, xsv, pbuf,
              sem0, sem1, lsem0, lsem1):
        wid = lax.axis_index("s") * 2 + lax.axis_index("c")
        iota = lax.iota(jnp.int32, _LANES)
        zero = jnp.zeros((_LANES,), jnp.float32)
        row0 = wid * RPT
        esems = (sem0, sem1)
        lsems = (lsem0, lsem1)

        def start_emb(idx):
            rbg, esub = divmod(idx, 2)
            b, rb = divmod(rbg, nrb)
            slot = idx % 2
            r = row0 + rb * _RB
            return pltpu.async_copy(
                emb_hbm.at[b, pl.ds(esub * _EH, _EH), pl.ds(r, _RB), :],
                embv.at[slot], esems[slot])

        def start_lab(rbg):
            b, rb = divmod(rbg, nrb)
            slot = rbg % 2
            r = row0 + rb * _RB
            return pltpu.async_copy(
                lab_hbm.at[b, pl.ds(r, _RB), :], labv.at[slot], lsems[slot])

        pltpu.sync_copy(ctr_hbm, ctrv)
        pltpu.sync_copy(csq_hbm, csqv)
        pltpu.sync_copy(zeros_hbm.at[pl.ds(0, HACC)], hacc)

        epend = {0: start_emb(0)}
        lpend = {0: start_lab(0)}
        for idx in range(nflat):
            rbg, esub = divmod(idx, 2)
            b, _rb = divmod(rbg, nrb)
            slot = idx % 2
            lslot = rbg % 2
            epend.pop(idx).wait()
            if esub == 0:
                lpend.pop(rbg).wait()
            if idx + 1 < nflat:
                epend[idx + 1] = start_emb(idx + 1)
                if (idx + 1) % 2 == 0 and rbg + 1 < Bsz * nrb:
                    lpend[rbg + 1] = start_lab(rbg + 1)

            if esub == 0:

                @plsc.parallel_loop(0, G, unroll=unroll)
                def body0(g):
                    r = lax.shift_right_logical(g, 5)
                    col = lax.shift_left(lax.bitwise_and(g, 31), 4)
                    lab = labv[lslot, r, pl.ds(col, _LANES)]
                    lab_e = lab * _E + b * (_NL * _E)
                    cx = zero
                    xs = zero
                    for e in range(_EH):
                        x = embv[slot, e, r, pl.ds(col, _LANES)]
                        ce = plsc.load_gather(ctrv, [lab_e + e])
                        cx = cx + ce * x
                        xs = xs + x * x
                    p = lax.shift_left(g, 4)
                    cxv[pl.ds(p, _LANES)] = cx
                    xsv[pl.ds(p, _LANES)] = xs
            else:

                @plsc.parallel_loop(0, G, unroll=unroll)
                def body1(g):
                    r = lax.shift_right_logical(g, 5)
                    col = lax.shift_left(lax.bitwise_and(g, 31), 4)
                    lab = labv[lslot, r, pl.ds(col, _LANES)]
                    lab_e = lab * _E + b * (_NL * _E)
                    p = lax.shift_left(g, 4)
                    cx = cxv[pl.ds(p, _LANES)]
                    xs = xsv[pl.ds(p, _LANES)]
                    for e in range(_EH):
                        x = embv[slot, e, r, pl.ds(col, _LANES)]
                        ce = plsc.load_gather(ctrv, [lab_e + _EH + e])
                        cx = cx + ce * x
                        xs = xs + x * x
                    cq = plsc.load_gather(csqv, [lab + b * _NL])
                    d2 = jnp.maximum(xs - 2.0 * cx + cq, 1e-12)
                    dd = d2 * _nrsqrt(d2)
                    h = jnp.maximum(dd - _DELTA_VAR, 0.0)
                    plsc.addupdate_scatter(
                        hacc, [lab * _LANES + iota + b * (_NL * _LANES)],
                        h * h)

        pltpu.sync_copy(hacc, hout_hbm.at[wid])

        @pl.when(wid == 0)
        def _pairs():
            for b in range(Bsz):
                pltpu.sync_copy(ctrT_hbm.at[b], ctrTv.at[b])
                pltpu.sync_copy(prs_hbm.at[b], prsv.at[b])
            for b in range(Bsz):
                prs = prsv[b, :]

                def pair_body(i, carry):
                    pairv, npv = carry
                    bi = jnp.broadcast_to(i, (_LANES,))
                    ssq = zero
                    bi_e = bi * _E + b * (_NL * _E)
                    for e in range(_E):
                        ce = plsc.load_gather(ctrv, [bi_e + e])
                        t = ctrTv[b, e, :] - ce
                        ssq = ssq + t * t
                    pi = plsc.load_gather(
                        prsv, [jnp.broadcast_to(b, (_LANES,)), bi])
                    m = jnp.where(iota > bi, 1.0, 0.0) * prs * pi
                    pd2 = jnp.maximum(ssq, 1e-12)
                    pd = pd2 * _nrsqrt(pd2)
                    hd = jnp.maximum(2.0 * _DELTA_DST - pd, 0.0)
                    return (pairv + hd * hd * m, npv + m)

                pairv, npv = lax.fori_loop(1, _NL, pair_body, (zero, zero))
                cq = jnp.maximum(csqv[pl.ds(b * _NL, _NL)], 1e-12)
                pbuf[0, :] = pairv
                pbuf[1, :] = npv
                pbuf[2, :] = cq * _nrsqrt(cq) * prs
                pltpu.sync_copy(pbuf, pout_hbm.at[b])

    return pass2


def kernel(embedding, ins_label):
    Bsz = embedding.shape[0]
    H = embedding.shape[2]
    emb = embedding.astype(jnp.float32)
    lab = ins_label.astype(jnp.int32)
    zeros = jnp.zeros((Bsz * _E * _NL * _LANES,), jnp.float32)

    sums_p, cnt_p = _make_pass1(Bsz, H, 8)(emb, lab, zeros)
    sums = sums_p.reshape(_NW, Bsz, _E, _NL, _LANES).sum(axis=(0, 4))
    counts = cnt_p.reshape(_NW, Bsz, _NL, _LANES).sum(axis=(0, 3))
    ids = jnp.arange(_NL)
    present = (counts > 0) & (ids[None, :] >= 1)
    presentf = present.astype(jnp.float32)
    safe = jnp.where(present, counts, 1.0)
    ctrT = jnp.where(present[:, None, :], sums / safe[:, None, :], 0.0)
    ctr = jnp.transpose(ctrT, (0, 2, 1)).reshape(Bsz * _NL * _E)
    csq = (ctrT ** 2).sum(axis=1)

    hpart, pout = _make_pass2(Bsz, H, 4)(
        emb, lab, zeros, ctr, ctrT, csq.reshape(Bsz * _NL), presentf)
    hsum = hpart.reshape(_NW, Bsz, _NL, _LANES).sum(axis=(0, 3))
    n_inst = presentf.sum(axis=1)
    loss_var_b = (jnp.where(present, hsum / safe, 0.0).sum(axis=1)
                  / jnp.maximum(n_inst, 1.0))
    pairsum = pout[:, 0, :].sum(axis=1)
    npairs = pout[:, 1, :].sum(axis=1)
    regsum = pout[:, 2, :].sum(axis=1)
    loss_dst_b = pairsum / jnp.maximum(npairs, 1.0)
    loss_reg_b = regsum / jnp.maximum(n_inst, 1.0)
    has = (n_inst > 0).astype(jnp.float32)
    denom = jnp.maximum(has.sum(), 1.0)
    lv = (has * loss_var_b).sum() / denom
    ld = (has * loss_dst_b).sum() / denom
    lr = (has * loss_reg_b).sum() / denom
    total = _A * lv + _B * ld + _R * lr
    return (total, lv, ld, lr)


# speedup vs baseline: 1.1603x; 1.1603x over previous
import functools

import jax
import jax.numpy as jnp
from jax import lax
from jax.experimental import pallas as pl
from jax.experimental.pallas import tpu as pltpu
from jax.experimental.pallas import tpu_sc as plsc

_DELTA_VAR = 0.5
_DELTA_DST = 1.5
_A = 1.0
_B = 1.0
_R = 0.001
_NL = 16
_E = 16
_EH = 8
_LANES = 16
_NW = 32
_RB = 8
_W = 512


def _nrsqrt(x):
    i = lax.bitcast_convert_type(x, jnp.int32)
    i = jnp.int32(0x5F3759DF) - lax.shift_right_arithmetic(i, 1)
    y = lax.bitcast_convert_type(i, jnp.float32)
    for _ in range(3):
        y = y * (1.5 - 0.5 * x * y * y)
    return y


@functools.lru_cache(maxsize=None)
def _make_pass1(Bsz, H, unroll):
    RPT = H // _NW
    nrb = RPT // _RB
    G = _RB * _W // _LANES
    nflat = Bsz * nrb * 2
    ACC = Bsz * _E * _NL * _LANES
    CNT = Bsz * _NL * _LANES
    mesh = plsc.VectorSubcoreMesh(core_axis_name="c", subcore_axis_name="s")
    out_types = (
        jax.ShapeDtypeStruct((_NW, ACC), jnp.float32),
        jax.ShapeDtypeStruct((_NW, CNT), jnp.float32),
    )

    @functools.partial(
        pl.kernel,
        out_type=out_types,
        mesh=mesh,
        compiler_params=pltpu.CompilerParams(needs_layout_passes=False),
        scratch_types=[
            pltpu.VMEM((2, _EH, _RB, _W), jnp.float32),
            pltpu.VMEM((2, _RB, _W), jnp.int32),
            pltpu.VMEM((ACC,), jnp.float32),
            pltpu.VMEM((CNT,), jnp.float32),
            pltpu.SemaphoreType.DMA,
            pltpu.SemaphoreType.DMA,
            pltpu.SemaphoreType.DMA,
            pltpu.SemaphoreType.DMA,
        ],
    )
    def pass1(emb_hbm, lab_hbm, zeros_hbm, sums_hbm, cnt_hbm,
              embv, labv, acc, cnt, sem0, sem1, lsem0, lsem1):
        wid = lax.axis_index("s") * 2 + lax.axis_index("c")
        iota = lax.iota(jnp.int32, _LANES)
        ones = jnp.ones((_LANES,), jnp.float32)
        row0 = wid * RPT
        esems = (sem0, sem1)
        lsems = (lsem0, lsem1)

        def start_emb(idx):
            rbg, esub = divmod(idx, 2)
            b, rb = divmod(rbg, nrb)
            slot = idx % 2
            r = row0 + rb * _RB
            return pltpu.async_copy(
                emb_hbm.at[b, pl.ds(esub * _EH, _EH), pl.ds(r, _RB), :],
                embv.at[slot], esems[slot])

        def start_lab(rbg):
            b, rb = divmod(rbg, nrb)
            slot = rbg % 2
            r = row0 + rb * _RB
            return pltpu.async_copy(
                lab_hbm.at[b, pl.ds(r, _RB), :], labv.at[slot], lsems[slot])

        pltpu.sync_copy(zeros_hbm.at[pl.ds(0, ACC)], acc)
        pltpu.sync_copy(zeros_hbm.at[pl.ds(0, CNT)], cnt)

        epend = {0: start_emb(0)}
        lpend = {0: start_lab(0)}
        for idx in range(nflat):
            rbg, esub = divmod(idx, 2)
            b, _rb = divmod(rbg, nrb)
            slot = idx % 2
            lslot = rbg % 2
            epend.pop(idx).wait()
            if esub == 0:
                lpend.pop(rbg).wait()
            if idx + 1 < nflat:
                epend[idx + 1] = start_emb(idx + 1)
                if (idx + 1) % 2 == 0 and rbg + 1 < Bsz * nrb:
                    lpend[rbg + 1] = start_lab(rbg + 1)

            @plsc.parallel_loop(0, G, unroll=unroll)
            def body(g):
                r = lax.shift_right_logical(g, 5)
                col = lax.shift_left(lax.bitwise_and(g, 31), 4)
                lab = labv[lslot, r, pl.ds(col, _LANES)]
                lane_base = lab * _LANES + iota + b * (_E * _NL * _LANES)
                for e in range(_EH):
                    x = embv[slot, e, r, pl.ds(col, _LANES)]
                    plsc.addupdate_scatter(
                        acc,
                        [lane_base + (esub * _EH + e) * (_NL * _LANES)], x)
                if esub == 0:
                    plsc.addupdate_scatter(
                        cnt, [lab * _LANES + iota + b * (_NL * _LANES)], ones)

        pltpu.sync_copy(acc, sums_hbm.at[wid])
        pltpu.sync_copy(cnt, cnt_hbm.at[wid])

    return pass1


@functools.lru_cache(maxsize=None)
def _make_pass2(Bsz, H, unroll):
    RPT = H // _NW
    nrb = RPT // _RB
    G = _RB * _W // _LANES
    CPX = _RB * _W
    nflat = Bsz * nrb * 2
    HACC = Bsz * _NL * _LANES
    mesh = plsc.VectorSubcoreMesh(core_axis_name="c", subcore_axis_name="s")
    out_types = (
        jax.ShapeDtypeStruct((_NW, HACC), jnp.float32),
        jax.ShapeDtypeStruct((Bsz, 3, _LANES), jnp.float32),
    )

    @functools.partial(
        pl.kernel,
        out_type=out_types,
        mesh=mesh,
        compiler_params=pltpu.CompilerParams(needs_layout_passes=False),
        scratch_types=[
            pltpu.VMEM((2, _EH, _RB, _W), jnp.float32),
            pltpu.VMEM((2, _RB, _W), jnp.int32),
            pltpu.VMEM((Bsz * _NL * _E,), jnp.float32),
            pltpu.VMEM((Bsz, _E, _NL), jnp.float32),
            pltpu.VMEM((Bsz * _NL,), jnp.float32),
            pltpu.VMEM((Bsz, _NL), jnp.float32),
            pltpu.VMEM((HACC,), jnp.float32),
            pltpu.VMEM((CPX,), jnp.float32),
            pltpu.VMEM((3, _LANES), jnp.float32),
            pltpu.SemaphoreType.DMA,
            pltpu.SemaphoreType.DMA,
            pltpu.SemaphoreType.DMA,
            pltpu.SemaphoreType.DMA,
        ],
    )
    def pass2(emb_hbm, lab_hbm, zeros_hbm, ctr_hbm, ctrT_hbm, csq_hbm,
              prs_hbm, hout_hbm, pout_hbm,
              embv, labv, ctrv, ctrTv, csqv, prsv, hacc, cxv, pbuf,
              sem0, sem1, lsem0, lsem1):
        wid = lax.axis_index("s") * 2 + lax.axis_index("c")
        iota = lax.iota(jnp.int32, _LANES)
        zero = jnp.zeros((_LANES,), jnp.float32)
        row0 = wid * RPT
        esems = (sem0, sem1)
        lsems = (lsem0, lsem1)

        def start_emb(idx):
            rbg, esub = divmod(idx, 2)
            b, rb = divmod(rbg, nrb)
            slot = idx % 2
            r = row0 + rb * _RB
            return pltpu.async_copy(
                emb_hbm.at[b, pl.ds(esub * _EH, _EH), pl.ds(r, _RB), :],
                embv.at[slot], esems[slot])

        def start_lab(rbg):
            b, rb = divmod(rbg, nrb)
            slot = rbg % 2
            r = row0 + rb * _RB
            return pltpu.async_copy(
                lab_hbm.at[b, pl.ds(r, _RB), :], labv.at[slot], lsems[slot])

        pltpu.sync_copy(ctr_hbm, ctrv)
        pltpu.sync_copy(csq_hbm, csqv)
        pltpu.sync_copy(zeros_hbm.at[pl.ds(0, HACC)], hacc)

        epend = {0: start_emb(0)}
        lpend = {0: start_lab(0)}
        for idx in range(nflat):
            rbg, esub = divmod(idx, 2)
            b, _rb = divmod(rbg, nrb)
            slot = idx % 2
            lslot = rbg % 2
            epend.pop(idx).wait()
            if esub == 0:
                lpend.pop(rbg).wait()
            if idx + 1 < nflat:
                epend[idx + 1] = start_emb(idx + 1)
                if (idx + 1) % 2 == 0 and rbg + 1 < Bsz * nrb:
                    lpend[rbg + 1] = start_lab(rbg + 1)

            if esub == 0:

                @plsc.parallel_loop(0, G, unroll=unroll)
                def body0(g):
                    r = lax.shift_right_logical(g, 5)
                    col = lax.shift_left(lax.bitwise_and(g, 31), 4)
                    lab = labv[lslot, r, pl.ds(col, _LANES)]
                    lab_e = lab * _E + b * (_NL * _E)
                    d2 = zero
                    for e in range(_EH):
                        x = embv[slot, e, r, pl.ds(col, _LANES)]
                        ce = plsc.load_gather(ctrv, [lab_e + e])
                        t = x - ce
                        d2 = d2 + t * t
                    p = lax.shift_left(g, 4)
                    cxv[pl.ds(p, _LANES)] = d2
            else:

                @plsc.parallel_loop(0, G, unroll=unroll)
                def body1(g):
                    r = lax.shift_right_logical(g, 5)
                    col = lax.shift_left(lax.bitwise_and(g, 31), 4)
                    lab = labv[lslot, r, pl.ds(col, _LANES)]
                    lab_e = lab * _E + b * (_NL * _E)
                    p = lax.shift_left(g, 4)
                    d2 = cxv[pl.ds(p, _LANES)]
                    for e in range(_EH):
                        x = embv[slot, e, r, pl.ds(col, _LANES)]
                        ce = plsc.load_gather(ctrv, [lab_e + _EH + e])
                        t = x - ce
                        d2 = d2 + t * t
                    d2 = jnp.maximum(d2, 1e-12)
                    dd = d2 * _nrsqrt(d2)
                    h = jnp.maximum(dd - _DELTA_VAR, 0.0)
                    plsc.addupdate_scatter(
                        hacc, [lab * _LANES + iota + b * (_NL * _LANES)],
                        h * h)

        pltpu.sync_copy(hacc, hout_hbm.at[wid])

        @pl.when(wid == 0)
        def _pairs():
            for b in range(Bsz):
                pltpu.sync_copy(ctrT_hbm.at[b], ctrTv.at[b])
                pltpu.sync_copy(prs_hbm.at[b], prsv.at[b])
            for b in range(Bsz):
                prs = prsv[b, :]

                def pair_body(i, carry):
                    pairv, npv = carry
                    bi = jnp.broadcast_to(i, (_LANES,))
                    ssq = zero
                    bi_e = bi * _E + b * (_NL * _E)
                    for e in range(_E):
                        ce = plsc.load_gather(ctrv, [bi_e + e])
                        t = ctrTv[b, e, :] - ce
                        ssq = ssq + t * t
                    pi = plsc.load_gather(
                        prsv, [jnp.broadcast_to(b, (_LANES,)), bi])
                    m = jnp.where(iota > bi, 1.0, 0.0) * prs * pi
                    pd2 = jnp.maximum(ssq, 1e-12)
                    pd = pd2 * _nrsqrt(pd2)
                    hd = jnp.maximum(2.0 * _DELTA_DST - pd, 0.0)
                    return (pairv + hd * hd * m, npv + m)

                pairv, npv = lax.fori_loop(1, _NL, pair_body, (zero, zero))
                cq = jnp.maximum(csqv[pl.ds(b * _NL, _NL)], 1e-12)
                pbuf[0, :] = pairv
                pbuf[1, :] = npv
                pbuf[2, :] = cq * _nrsqrt(cq) * prs
                pltpu.sync_copy(pbuf, pout_hbm.at[b])

    return pass2


def kernel(embedding, ins_label):
    Bsz = embedding.shape[0]
    H = embedding.shape[2]
    emb = embedding.astype(jnp.float32)
    lab = ins_label.astype(jnp.int32)
    zeros = jnp.zeros((Bsz * _E * _NL * _LANES,), jnp.float32)

    sums_p, cnt_p = _make_pass1(Bsz, H, 4)(emb, lab, zeros)
    sums = sums_p.reshape(_NW, Bsz, _E, _NL, _LANES).sum(axis=(0, 4))
    counts = cnt_p.reshape(_NW, Bsz, _NL, _LANES).sum(axis=(0, 3))
    ids = jnp.arange(_NL)
    present = (counts > 0) & (ids[None, :] >= 1)
    presentf = present.astype(jnp.float32)
    safe = jnp.where(present, counts, 1.0)
    ctrT = jnp.where(present[:, None, :], sums / safe[:, None, :], 0.0)
    ctr = jnp.transpose(ctrT, (0, 2, 1)).reshape(Bsz * _NL * _E)
    csq = (ctrT ** 2).sum(axis=1)

    hpart, pout = _make_pass2(Bsz, H, 2)(
        emb, lab, zeros, ctr, ctrT, csq.reshape(Bsz * _NL), presentf)
    hsum = hpart.reshape(_NW, Bsz, _NL, _LANES).sum(axis=(0, 3))
    n_inst = presentf.sum(axis=1)
    loss_var_b = (jnp.where(present, hsum / safe, 0.0).sum(axis=1)
                  / jnp.maximum(n_inst, 1.0))
    pairsum = pout[:, 0, :].sum(axis=1)
    npairs = pout[:, 1, :].sum(axis=1)
    regsum = pout[:, 2, :].sum(axis=1)
    loss_dst_b = pairsum / jnp.maximum(npairs, 1.0)
    loss_reg_b = regsum / jnp.maximum(n_inst, 1.0)
    has = (n_inst > 0).astype(jnp.float32)
    denom = jnp.maximum(has.sum(), 1.0)
    lv = (has * loss_var_b).sum() / denom
    ld = (has * loss_dst_b).sum() / denom
    lr = (has * loss_reg_b).sum() / denom
    total = _A * lv + _B * ld + _R * lr
    return (total, lv, ld, lr)


# speedup vs baseline: 1.6175x; 1.3941x over previous
import functools

import jax
import jax.numpy as jnp
from jax import lax
from jax.experimental import pallas as pl
from jax.experimental.pallas import tpu as pltpu
from jax.experimental.pallas import tpu_sc as plsc

_DELTA_VAR = 0.5
_DELTA_DST = 1.5
_A = 1.0
_B = 1.0
_R = 0.001
_NL = 16
_E = 16
_EH = 8
_LANES = 16
_NW = 32
_RB = 8
_W = 512


def _nrsqrt(x):
    i = lax.bitcast_convert_type(x, jnp.int32)
    i = jnp.int32(0x5F3759DF) - lax.shift_right_arithmetic(i, 1)
    y = lax.bitcast_convert_type(i, jnp.float32)
    for _ in range(3):
        y = y * (1.5 - 0.5 * x * y * y)
    return y


@functools.lru_cache(maxsize=None)
def _make_pass1(Bsz, H, unroll):
    RPT = H // _NW
    nrb = RPT // _RB
    G = _RB * _W // _LANES
    nflat = Bsz * nrb * 2
    ACC = Bsz * _E * _NL * _LANES
    CNT = Bsz * _NL * _LANES
    mesh = plsc.VectorSubcoreMesh(core_axis_name="c", subcore_axis_name="s")
    out_types = (
        jax.ShapeDtypeStruct((_NW, ACC), jnp.float32),
        jax.ShapeDtypeStruct((_NW, CNT), jnp.float32),
    )

    @functools.partial(
        pl.kernel,
        out_type=out_types,
        mesh=mesh,
        compiler_params=pltpu.CompilerParams(needs_layout_passes=False),
        scratch_types=[
            pltpu.VMEM((2, _EH, _RB, _W), jnp.float32),
            pltpu.VMEM((2, _RB, _W), jnp.int32),
            pltpu.VMEM((ACC,), jnp.float32),
            pltpu.VMEM((CNT,), jnp.float32),
            pltpu.SemaphoreType.DMA,
            pltpu.SemaphoreType.DMA,
            pltpu.SemaphoreType.DMA,
            pltpu.SemaphoreType.DMA,
        ],
    )
    def pass1(emb_hbm, lab_hbm, zeros_hbm, sums_hbm, cnt_hbm,
              embv, labv, acc, cnt, sem0, sem1, lsem0, lsem1):
        wid = lax.axis_index("s") * 2 + lax.axis_index("c")
        iota = lax.iota(jnp.int32, _LANES)
        ones = jnp.ones((_LANES,), jnp.float32)
        row0 = wid * RPT
        esems = (sem0, sem1)
        lsems = (lsem0, lsem1)

        def start_emb(idx):
            rbg, esub = divmod(idx, 2)
            b, rb = divmod(rbg, nrb)
            slot = idx % 2
            r = row0 + rb * _RB
            return pltpu.async_copy(
                emb_hbm.at[b, pl.ds(esub * _EH, _EH), pl.ds(r, _RB), :],
                embv.at[slot], esems[slot])

        def start_lab(rbg):
            b, rb = divmod(rbg, nrb)
            slot = rbg % 2
            r = row0 + rb * _RB
            return pltpu.async_copy(
                lab_hbm.at[b, pl.ds(r, _RB), :], labv.at[slot], lsems[slot])

        pltpu.sync_copy(zeros_hbm.at[pl.ds(0, ACC)], acc)
        pltpu.sync_copy(zeros_hbm.at[pl.ds(0, CNT)], cnt)

        epend = {0: start_emb(0)}
        lpend = {0: start_lab(0)}
        for idx in range(nflat):
            rbg, esub = divmod(idx, 2)
            b, _rb = divmod(rbg, nrb)
            slot = idx % 2
            lslot = rbg % 2
            epend.pop(idx).wait()
            if esub == 0:
                lpend.pop(rbg).wait()
            if idx + 1 < nflat:
                epend[idx + 1] = start_emb(idx + 1)
                if (idx + 1) % 2 == 0 and rbg + 1 < Bsz * nrb:
                    lpend[rbg + 1] = start_lab(rbg + 1)

            @plsc.parallel_loop(0, G, unroll=unroll)
            def body(g):
                r = lax.shift_right_logical(g, 5)
                col = lax.shift_left(lax.bitwise_and(g, 31), 4)
                lab = labv[lslot, r, pl.ds(col, _LANES)]
                lane_base = lab * _LANES + iota + b * (_E * _NL * _LANES)
                for e in range(_EH):
                    x = embv[slot, e, r, pl.ds(col, _LANES)]
                    plsc.addupdate_scatter(
                        acc,
                        [lane_base + (esub * _EH + e) * (_NL * _LANES)], x)
                if esub == 0:
                    plsc.addupdate_scatter(
                        cnt, [lab * _LANES + iota + b * (_NL * _LANES)], ones)

        pltpu.sync_copy(acc, sums_hbm.at[wid])
        pltpu.sync_copy(cnt, cnt_hbm.at[wid])

    return pass1


@functools.lru_cache(maxsize=None)
def _make_pass2(Bsz, H, unroll):
    RPT = H // _NW
    nrb = RPT // _RB
    G = _RB * _W // _LANES
    CPX = _RB * _W
    nflat = Bsz * nrb * 2
    HACC = Bsz * _NL * _LANES
    mesh = plsc.VectorSubcoreMesh(core_axis_name="c", subcore_axis_name="s")
    out_types = (
        jax.ShapeDtypeStruct((_NW, HACC), jnp.float32),
        jax.ShapeDtypeStruct((Bsz, 3, _LANES), jnp.float32),
    )

    @functools.partial(
        pl.kernel,
        out_type=out_types,
        mesh=mesh,
        compiler_params=pltpu.CompilerParams(needs_layout_passes=False),
        scratch_types=[
            pltpu.VMEM((2, _EH, _RB, _W), jnp.float32),
            pltpu.VMEM((2, _RB, _W), jnp.int32),
            pltpu.VMEM((Bsz * _NL * _E * _LANES,), jnp.float32),
            pltpu.VMEM((Bsz, _E, _NL), jnp.float32),
            pltpu.VMEM((Bsz * _NL,), jnp.float32),
            pltpu.VMEM((Bsz, _NL), jnp.float32),
            pltpu.VMEM((HACC,), jnp.float32),
            pltpu.VMEM((CPX,), jnp.float32),
            pltpu.VMEM((3, _LANES), jnp.float32),
            pltpu.SemaphoreType.DMA,
            pltpu.SemaphoreType.DMA,
            pltpu.SemaphoreType.DMA,
            pltpu.SemaphoreType.DMA,
        ],
    )
    def pass2(emb_hbm, lab_hbm, zeros_hbm, ctr_hbm, ctrT_hbm, csq_hbm,
              prs_hbm, hout_hbm, pout_hbm,
              embv, labv, ctrv, ctrTv, csqv, prsv, hacc, cxv, pbuf,
              sem0, sem1, lsem0, lsem1):
        wid = lax.axis_index("s") * 2 + lax.axis_index("c")
        iota = lax.iota(jnp.int32, _LANES)
        zero = jnp.zeros((_LANES,), jnp.float32)
        row0 = wid * RPT
        esems = (sem0, sem1)
        lsems = (lsem0, lsem1)

        def start_emb(idx):
            rbg, esub = divmod(idx, 2)
            b, rb = divmod(rbg, nrb)
            slot = idx % 2
            r = row0 + rb * _RB
            return pltpu.async_copy(
                emb_hbm.at[b, pl.ds(esub * _EH, _EH), pl.ds(r, _RB), :],
                embv.at[slot], esems[slot])

        def start_lab(rbg):
            b, rb = divmod(rbg, nrb)
            slot = rbg % 2
            r = row0 + rb * _RB
            return pltpu.async_copy(
                lab_hbm.at[b, pl.ds(r, _RB), :], labv.at[slot], lsems[slot])

        pltpu.sync_copy(ctr_hbm, ctrv)
        pltpu.sync_copy(csq_hbm, csqv)
        pltpu.sync_copy(zeros_hbm.at[pl.ds(0, HACC)], hacc)

        epend = {0: start_emb(0)}
        lpend = {0: start_lab(0)}
        for idx in range(nflat):
            rbg, esub = divmod(idx, 2)
            b, _rb = divmod(rbg, nrb)
            slot = idx % 2
            lslot = rbg % 2
            epend.pop(idx).wait()
            if esub == 0:
                lpend.pop(rbg).wait()
            if idx + 1 < nflat:
                epend[idx + 1] = start_emb(idx + 1)
                if (idx + 1) % 2 == 0 and rbg + 1 < Bsz * nrb:
                    lpend[rbg + 1] = start_lab(rbg + 1)

            if esub == 0:

                @plsc.parallel_loop(0, G, unroll=unroll)
                def body0(g):
                    r = lax.shift_right_logical(g, 5)
                    col = lax.shift_left(lax.bitwise_and(g, 31), 4)
                    lab = labv[lslot, r, pl.ds(col, _LANES)]
                    lab_e = (lab * _E + b * (_NL * _E)) * _LANES + iota
                    d2 = zero
                    for e in range(_EH):
                        x = embv[slot, e, r, pl.ds(col, _LANES)]
                        ce = plsc.load_gather(ctrv, [lab_e + e * _LANES])
                        t = x - ce
                        d2 = d2 + t * t
                    p = lax.shift_left(g, 4)
                    cxv[pl.ds(p, _LANES)] = d2
            else:

                @plsc.parallel_loop(0, G, unroll=unroll)
                def body1(g):
                    r = lax.shift_right_logical(g, 5)
                    col = lax.shift_left(lax.bitwise_and(g, 31), 4)
                    lab = labv[lslot, r, pl.ds(col, _LANES)]
                    lab_e = (lab * _E + b * (_NL * _E)) * _LANES + iota
                    p = lax.shift_left(g, 4)
                    d2 = cxv[pl.ds(p, _LANES)]
                    for e in range(_EH):
                        x = embv[slot, e, r, pl.ds(col, _LANES)]
                        ce = plsc.load_gather(ctrv, [lab_e + (_EH + e) * _LANES])
                        t = x - ce
                        d2 = d2 + t * t
                    d2 = jnp.maximum(d2, 1e-12)
                    dd = d2 * _nrsqrt(d2)
                    h = jnp.maximum(dd - _DELTA_VAR, 0.0)
                    plsc.addupdate_scatter(
                        hacc, [lab * _LANES + iota + b * (_NL * _LANES)],
                        h * h)

        pltpu.sync_copy(hacc, hout_hbm.at[wid])

        @pl.when(wid == 0)
        def _pairs():
            for b in range(Bsz):
                pltpu.sync_copy(ctrT_hbm.at[b], ctrTv.at[b])
                pltpu.sync_copy(prs_hbm.at[b], prsv.at[b])
            for b in range(Bsz):
                prs = prsv[b, :]

                def pair_body(i, carry):
                    pairv, npv = carry
                    bi = jnp.broadcast_to(i, (_LANES,))
                    ssq = zero
                    bi_e = (bi * _E + b * (_NL * _E)) * _LANES + iota
                    for e in range(_E):
                        ce = plsc.load_gather(ctrv, [bi_e + e * _LANES])
                        t = ctrTv[b, e, :] - ce
                        ssq = ssq + t * t
                    pi = plsc.load_gather(
                        prsv, [jnp.broadcast_to(b, (_LANES,)), bi])
                    m = jnp.where(iota > bi, 1.0, 0.0) * prs * pi
                    pd2 = jnp.maximum(ssq, 1e-12)
                    pd = pd2 * _nrsqrt(pd2)
                    hd = jnp.maximum(2.0 * _DELTA_DST - pd, 0.0)
                    return (pairv + hd * hd * m, npv + m)

                pairv, npv = lax.fori_loop(1, _NL, pair_body, (zero, zero))
                cq = jnp.maximum(csqv[pl.ds(b * _NL, _NL)], 1e-12)
                pbuf[0, :] = pairv
                pbuf[1, :] = npv
                pbuf[2, :] = cq * _nrsqrt(cq) * prs
                pltpu.sync_copy(pbuf, pout_hbm.at[b])

    return pass2


def kernel(embedding, ins_label):
    Bsz = embedding.shape[0]
    H = embedding.shape[2]
    emb = embedding.astype(jnp.float32)
    lab = ins_label.astype(jnp.int32)
    zeros = jnp.zeros((Bsz * _E * _NL * _LANES,), jnp.float32)

    sums_p, cnt_p = _make_pass1(Bsz, H, 4)(emb, lab, zeros)
    sums = sums_p.reshape(_NW, Bsz, _E, _NL, _LANES).sum(axis=(0, 4))
    counts = cnt_p.reshape(_NW, Bsz, _NL, _LANES).sum(axis=(0, 3))
    ids = jnp.arange(_NL)
    present = (counts > 0) & (ids[None, :] >= 1)
    presentf = present.astype(jnp.float32)
    safe = jnp.where(present, counts, 1.0)
    ctrT = jnp.where(present[:, None, :], sums / safe[:, None, :], 0.0)
    ctr = jnp.broadcast_to(
        jnp.transpose(ctrT, (0, 2, 1)).reshape(Bsz * _NL * _E)[:, None],
        (Bsz * _NL * _E, _LANES)).reshape(-1)
    csq = (ctrT ** 2).sum(axis=1)

    hpart, pout = _make_pass2(Bsz, H, 2)(
        emb, lab, zeros, ctr, ctrT, csq.reshape(Bsz * _NL), presentf)
    hsum = hpart.reshape(_NW, Bsz, _NL, _LANES).sum(axis=(0, 3))
    n_inst = presentf.sum(axis=1)
    loss_var_b = (jnp.where(present, hsum / safe, 0.0).sum(axis=1)
                  / jnp.maximum(n_inst, 1.0))
    pairsum = pout[:, 0, :].sum(axis=1)
    npairs = pout[:, 1, :].sum(axis=1)
    regsum = pout[:, 2, :].sum(axis=1)
    loss_dst_b = pairsum / jnp.maximum(npairs, 1.0)
    loss_reg_b = regsum / jnp.maximum(n_inst, 1.0)
    has = (n_inst > 0).astype(jnp.float32)
    denom = jnp.maximum(has.sum(), 1.0)
    lv = (has * loss_var_b).sum() / denom
    ld = (has * loss_dst_b).sum() / denom
    lr = (has * loss_reg_b).sum() / denom
    total = _A * lv + _B * ld + _R * lr
    return (total, lv, ld, lr)
